# baseline (speedup 1.0000x reference)
import functools

import jax
import jax.numpy as jnp
from jax import lax
from jax.experimental import pallas as pl
from jax.experimental.pallas import tpu as pltpu
from jax.experimental.pallas import tpu_sc as plsc

_L = 16
_RPS = 32


@jax.jit
def _gather(node, table):
    B, = node.shape
    V, D = table.shape
    table_sr = table.reshape(V // _RPS, _RPS * D)
    info = plsc.get_sparse_core_info()
    NC, NS = info.num_cores, info.num_subcores
    NW = NC * NS
    b_per_w = B // NW
    CH = 128
    n_ch = b_per_w // CH
    mesh = plsc.VectorSubcoreMesh(core_axis_name="c", subcore_axis_name="s")

    @functools.partial(
        pl.kernel,
        mesh=mesh,
        out_type=jax.ShapeDtypeStruct((B * D,), jnp.float32),
        scratch_types=[
            pltpu.VMEM((b_per_w,), jnp.int32),
            pltpu.VMEM((n_ch, CH), jnp.int32),
            pltpu.VMEM((b_per_w, _RPS * D), jnp.float32),
            pltpu.VMEM((b_per_w * D,), jnp.float32),
            pltpu.SemaphoreType.DMA,
        ],
        compiler_params=pltpu.CompilerParams(needs_layout_passes=False),
    )
    def k(node_hbm, table_hbm, out_hbm, nd_v, sr_v, rows_v, out_v, sem):
        wid = lax.axis_index("s") * NC + lax.axis_index("c")
        base = wid * b_per_w
        pltpu.sync_copy(node_hbm.at[pl.ds(base, b_per_w)], nd_v)

        for j in range(n_ch):
            for t in range(CH // _L):
                nd = nd_v[pl.ds(j * CH + t * _L, _L)]
                sr_v[j, pl.ds(t * _L, _L)] = nd >> 5

        copies = [
            pltpu.async_copy(
                table_hbm.at[sr_v.at[j]],
                rows_v.at[pl.ds(j * CH, CH)],
                sem,
            )
            for j in range(n_ch)
        ]
        for c in copies:
            c.wait()

        lane = lax.iota(jnp.int32, _L)
        bid0 = lane >> 2
        col0 = lane & 3
        for v in range(b_per_w * D // _L):
            bid = bid0 + (v * _L // D)
            nd = plsc.load_gather(nd_v, [bid])
            col = ((nd & (_RPS - 1)) << 2) + col0
            vals = plsc.load_gather(rows_v, [bid, col])
            out_v[pl.ds(v * _L, _L)] = vals

        pltpu.sync_copy(out_v, out_hbm.at[pl.ds(base * D, b_per_w * D)])

    return k(node, table_sr).reshape(B, D)


def kernel(node, table):
    return _gather(node.astype(jnp.int32), table)
